# Initial kernel scaffold; baseline (speedup 1.0000x reference)
import jax
import jax.numpy as jnp
from jax.experimental import pallas as pl


def kernel(inputs, W_router, W1, W2):
    raise NotImplementedError("write your pallas kernel here")


# speedup vs baseline: 1.0546x; 1.0546x over previous
import functools

import jax
import jax.numpy as jnp
from jax.experimental import pallas as pl
from jax.experimental.pallas import tpu as pltpu

N_EXP = 8
K = 2
D = 1024
F = 4096

TB = 512
FB = 1024


def _router_body(x_ref, wr_ref, logits_ref, sel_ref, cw_ref):
    x = x_ref[...]
    wr = wr_ref[...]
    logits = jax.lax.dot_general(
        x, wr, (((1,), (0,)), ((), ())), preferred_element_type=jnp.float32)
    logits_ref[...] = logits
    probs = jax.nn.softmax(logits, axis=1)
    i0 = jnp.argmax(probs, axis=1)
    neg = jnp.full_like(probs, -jnp.inf)
    lane = jax.lax.broadcasted_iota(jnp.int32, probs.shape, 1)
    probs_m = jnp.where(lane == i0[:, None], neg, probs)
    i1 = jnp.argmax(probs_m, axis=1)
    sel_ref[...] = jnp.stack([i0, i1], axis=1).astype(jnp.int32)
    keep = (lane == i0[:, None]) | (lane == i1[:, None])
    cw_ref[...] = jnp.where(keep, probs, 0.0)


def _expert_body(x_ref, w1_ref, w2_ref, cw_ref, out_ref):
    e = pl.program_id(1)
    f = pl.program_id(2)

    @pl.when((e == 0) & (f == 0))
    def _():
        out_ref[...] = jnp.zeros_like(out_ref)

    x = x_ref[...]
    h = jax.lax.dot_general(
        x, w1_ref[0], (((1,), (0,)), ((), ())),
        preferred_element_type=jnp.float32)
    h = jax.nn.gelu(h)
    out = jax.lax.dot_general(
        h, w2_ref[0], (((1,), (0,)), ((), ())),
        preferred_element_type=jnp.float32)
    lane = jax.lax.broadcasted_iota(jnp.int32, (TB, N_EXP), 1)
    w_col = jnp.sum(jnp.where(lane == e, cw_ref[...], 0.0), axis=1)
    out_ref[...] += w_col[:, None] * out


def kernel(inputs, W_router, W1, W2):
    xs = inputs.reshape(-1, D)
    T = xs.shape[0]
    n_tb = T // TB

    logits, sel, cw = pl.pallas_call(
        _router_body,
        grid=(n_tb,),
        in_specs=[
            pl.BlockSpec((TB, D), lambda t: (t, 0)),
            pl.BlockSpec((D, N_EXP), lambda t: (0, 0)),
        ],
        out_specs=[
            pl.BlockSpec((TB, N_EXP), lambda t: (t, 0)),
            pl.BlockSpec((TB, K), lambda t: (t, 0)),
            pl.BlockSpec((TB, N_EXP), lambda t: (t, 0)),
        ],
        out_shape=[
            jax.ShapeDtypeStruct((T, N_EXP), jnp.float32),
            jax.ShapeDtypeStruct((T, K), jnp.int32),
            jax.ShapeDtypeStruct((T, N_EXP), jnp.float32),
        ],
    )(xs, W_router)

    results = pl.pallas_call(
        _expert_body,
        grid=(n_tb, N_EXP, F // FB),
        in_specs=[
            pl.BlockSpec((TB, D), lambda t, e, f: (t, 0)),
            pl.BlockSpec((1, D, FB), lambda t, e, f: (e, 0, f)),
            pl.BlockSpec((1, FB, D), lambda t, e, f: (e, f, 0)),
            pl.BlockSpec((TB, N_EXP), lambda t, e, f: (t, 0)),
        ],
        out_specs=pl.BlockSpec((TB, D), lambda t, e, f: (t, 0)),
        out_shape=jax.ShapeDtypeStruct((T, D), jnp.float32),
        compiler_params=pltpu.CompilerParams(
            dimension_semantics=("parallel", "arbitrary", "arbitrary"),
        ),
    )(xs, W1, W2, cw)

    return (results.reshape(inputs.shape), logits, sel)


# speedup vs baseline: 1.5556x; 1.4750x over previous
import functools

import jax
import jax.numpy as jnp
from jax.experimental import pallas as pl
from jax.experimental.pallas import tpu as pltpu

N_EXP = 8
K = 2
D = 1024
F = 4096

TB = 512
BT = 256
CAP = 4096 * K + N_EXP * BT
NB = CAP // BT


def _router_body(x_ref, wr_ref, logits_ref, sel_ref, selT_ref, wT_ref):
    x = x_ref[...]
    logits = jax.lax.dot_general(
        x, wr_ref[...], (((1,), (0,)), ((), ())),
        preferred_element_type=jnp.float32)
    logits_ref[...] = logits
    probs = jax.nn.softmax(logits, axis=1)
    i0 = jnp.argmax(probs, axis=1)
    lane = jax.lax.broadcasted_iota(jnp.int32, probs.shape, 1)
    m0 = lane == i0[:, None]
    w0 = jnp.max(probs, axis=1)
    probs_m = jnp.where(m0, -jnp.inf, probs)
    i1 = jnp.argmax(probs_m, axis=1)
    w1 = jnp.max(probs_m, axis=1)
    sel_ref[...] = jnp.stack([i0, i1], axis=1).astype(jnp.int32)
    selT_ref[...] = jnp.stack([i0, i1], axis=0).astype(jnp.int32)
    wT_ref[...] = jnp.stack([w0, w1], axis=0)


def _cumsum_body(sel_ref, cexcl_ref, counts_ref, carry):
    t = pl.program_id(0)

    @pl.when(t == 0)
    def _():
        carry[...] = jnp.zeros_like(carry)

    sel = sel_ref[...]
    lane = jax.lax.broadcasted_iota(jnp.int32, (TB, N_EXP), 1)
    onehot = ((lane == sel[:, 0][:, None]) |
              (lane == sel[:, 1][:, None])).astype(jnp.float32)
    r = jax.lax.broadcasted_iota(jnp.int32, (TB, TB), 0)
    c = jax.lax.broadcasted_iota(jnp.int32, (TB, TB), 1)
    tril_strict = (c < r).astype(jnp.float32)
    local = jax.lax.dot_general(
        tril_strict, onehot, (((1,), (0,)), ((), ())),
        preferred_element_type=jnp.float32)
    cexcl_ref[...] = local + carry[...]
    carry[...] += jnp.sum(onehot, axis=0, keepdims=True)

    @pl.when(t == pl.num_programs(0) - 1)
    def _():
        counts_ref[...] = carry[...]


def _dispatch_body(counts_ref, cexcl_ref, sel_ref, posT_ref, be_ref):
    counts = counts_ref[...]
    aligned = jnp.ceil(counts / BT) * BT
    u = jax.lax.broadcasted_iota(jnp.int32, (N_EXP, N_EXP), 0)
    v = jax.lax.broadcasted_iota(jnp.int32, (N_EXP, N_EXP), 1)
    incl_tri = (u <= v).astype(jnp.float32)
    incl = jax.lax.dot_general(
        aligned, incl_tri, (((1,), (0,)), ((), ())),
        preferred_element_type=jnp.float32)
    excl = incl - aligned
    sel = sel_ref[...]
    cexcl = cexcl_ref[...]
    lane = jax.lax.broadcasted_iota(jnp.int32, (4096, N_EXP), 1)
    base = excl + cexcl
    p0 = jnp.sum(jnp.where(lane == sel[:, 0][:, None], base, 0.0), axis=1)
    p1 = jnp.sum(jnp.where(lane == sel[:, 1][:, None], base, 0.0), axis=1)
    posT_ref[...] = jnp.stack([p0, p1], axis=0).astype(jnp.int32)
    incl_sub = jnp.transpose(incl)
    bl = jax.lax.broadcasted_iota(jnp.int32, (N_EXP, 64), 1)
    m = ((bl * BT).astype(jnp.float32) >= incl_sub).astype(jnp.float32)
    be = jax.lax.dot_general(
        jnp.ones((1, N_EXP), jnp.float32), m, (((1,), (0,)), ((), ())),
        preferred_element_type=jnp.float32)
    be_ref[...] = jnp.minimum(be, N_EXP - 1).astype(jnp.int32)


FB = 2048
NF = F // FB


def _gmm_body(be_ref, x_ref, w1_ref, w2_ref, ws_ref, acc_ref, out_ref):
    f = pl.program_id(0)
    h = jax.lax.dot_general(
        x_ref[...], w1_ref[0], (((1,), (0,)), ((), ())),
        preferred_element_type=jnp.float32)
    h = jax.nn.gelu(h)
    out = jax.lax.dot_general(
        h, w2_ref[0], (((1,), (0,)), ((), ())),
        preferred_element_type=jnp.float32)
    contrib = out * ws_ref[...]

    @pl.when(f == 0)
    def _():
        out_ref[...] = contrib

    @pl.when(f != 0)
    def _():
        out_ref[...] = acc_ref[...] + contrib


def kernel(inputs, W_router, W1, W2):
    xs = inputs.reshape(-1, D)
    T = xs.shape[0]
    n_tb = T // TB

    logits, sel, selT, wT = pl.pallas_call(
        _router_body,
        grid=(n_tb,),
        in_specs=[
            pl.BlockSpec((TB, D), lambda t: (t, 0)),
            pl.BlockSpec((D, N_EXP), lambda t: (0, 0)),
        ],
        out_specs=[
            pl.BlockSpec((TB, N_EXP), lambda t: (t, 0)),
            pl.BlockSpec((TB, K), lambda t: (t, 0)),
            pl.BlockSpec((K, TB), lambda t: (0, t)),
            pl.BlockSpec((K, TB), lambda t: (0, t)),
        ],
        out_shape=[
            jax.ShapeDtypeStruct((T, N_EXP), jnp.float32),
            jax.ShapeDtypeStruct((T, K), jnp.int32),
            jax.ShapeDtypeStruct((K, T), jnp.int32),
            jax.ShapeDtypeStruct((K, T), jnp.float32),
        ],
    )(xs, W_router)

    cexcl, counts = pl.pallas_call(
        _cumsum_body,
        grid=(n_tb,),
        in_specs=[pl.BlockSpec((TB, K), lambda t: (t, 0))],
        out_specs=[
            pl.BlockSpec((TB, N_EXP), lambda t: (t, 0)),
            pl.BlockSpec((1, N_EXP), lambda t: (0, 0)),
        ],
        out_shape=[
            jax.ShapeDtypeStruct((T, N_EXP), jnp.float32),
            jax.ShapeDtypeStruct((1, N_EXP), jnp.float32),
        ],
        scratch_shapes=[pltpu.VMEM((1, N_EXP), jnp.float32)],
        compiler_params=pltpu.CompilerParams(
            dimension_semantics=("arbitrary",)),
    )(sel)

    posT, be = pl.pallas_call(
        _dispatch_body,
        in_specs=[
            pl.BlockSpec((1, N_EXP), lambda: (0, 0)),
            pl.BlockSpec((T, N_EXP), lambda: (0, 0)),
            pl.BlockSpec((T, K), lambda: (0, 0)),
        ],
        out_specs=[
            pl.BlockSpec((K, T), lambda: (0, 0)),
            pl.BlockSpec((1, 64), lambda: (0, 0)),
        ],
        out_shape=[
            jax.ShapeDtypeStruct((K, T), jnp.int32),
            jax.ShapeDtypeStruct((1, 64), jnp.int32),
        ],
    )(counts, cexcl, sel)

    pos_flat = posT.reshape(-1)
    tok_ids = jnp.tile(jnp.arange(T, dtype=jnp.int32), K)
    sorted_tok = jnp.zeros((CAP,), jnp.int32).at[pos_flat].set(tok_ids)
    sorted_w = jnp.zeros((CAP,), jnp.float32).at[pos_flat].set(wT.reshape(-1))
    x_sorted = xs[sorted_tok]

    block_expert = be[0, :NB]

    grid_spec = pltpu.PrefetchScalarGridSpec(
        num_scalar_prefetch=1,
        grid=(NF, NB),
        in_specs=[
            pl.BlockSpec((BT, D), lambda f, b, be_r: (b, 0)),
            pl.BlockSpec((1, D, FB), lambda f, b, be_r: (be_r[b], 0, f)),
            pl.BlockSpec((1, FB, D), lambda f, b, be_r: (be_r[b], f, 0)),
            pl.BlockSpec((BT, 1), lambda f, b, be_r: (b, 0)),
            pl.BlockSpec((BT, D), lambda f, b, be_r: (b, 0)),
        ],
        out_specs=pl.BlockSpec((BT, D), lambda f, b, be_r: (b, 0)),
    )
    acc_init = jnp.zeros((CAP, D), jnp.float32)
    out_sorted = pl.pallas_call(
        _gmm_body,
        grid_spec=grid_spec,
        out_shape=jax.ShapeDtypeStruct((CAP, D), jnp.float32),
        input_output_aliases={5: 0},
        compiler_params=pltpu.CompilerParams(
            dimension_semantics=("arbitrary", "arbitrary"),
            vmem_limit_bytes=60 * 1024 * 1024,
        ),
    )(block_expert, x_sorted, W1, W2, sorted_w.reshape(CAP, 1), acc_init)

    results = out_sorted[posT[0]] + out_sorted[posT[1]]

    return (results.reshape(inputs.shape), logits, sel)
